# Initial kernel scaffold; baseline (speedup 1.0000x reference)
import jax
import jax.numpy as jnp
from jax.experimental import pallas as pl


def kernel(x, edge_index, W1l, b1l, W1r, W2l, b2l, W2r):
    raise NotImplementedError("write your pallas kernel here")


# speedup vs baseline: 6.9869x; 6.9869x over previous
import functools

import jax
import jax.numpy as jnp
from jax import lax
from jax.experimental import pallas as pl
from jax.experimental.pallas import tpu as pltpu
from jax.experimental.pallas import tpu_sc as plsc

N_NODES = 10000
N_EDGES = 320000
D_IN = 128
D_HID = 16

NC = 2
NS = 16
NW = NC * NS
EPW = N_EDGES // NW
CH = 80
NCH = EPW // CH
SL_A = 624
SL_B = N_NODES - 15 * SL_A


def _tc_linear2(x, wl, wr):
    def body(x_ref, wl_ref, wr_ref, y_ref, p_ref):
        xv = x_ref[...]
        y_ref[...] = jnp.dot(xv, wl_ref[...], preferred_element_type=jnp.float32)
        p_ref[...] = jnp.dot(xv, wr_ref[...], preferred_element_type=jnp.float32)

    n = x.shape[0]
    return pl.pallas_call(
        body,
        out_shape=[
            jax.ShapeDtypeStruct((n, wl.shape[1]), jnp.float32),
            jax.ShapeDtypeStruct((n, wr.shape[1]), jnp.float32),
        ],
    )(x, wl, wr)


def _sc_aggregate1(src, dst, y1):
    mesh = plsc.VectorSubcoreMesh(core_axis_name="c", subcore_axis_name="s")

    @functools.partial(
        pl.kernel,
        out_type=[
            jax.ShapeDtypeStruct((NC * N_NODES, D_HID), jnp.float32),
            jax.ShapeDtypeStruct((NC * N_NODES,), jnp.float32),
        ],
        mesh=mesh,
        compiler_params=pltpu.CompilerParams(use_tc_tiling_on_sc=False),
        scratch_types=[
            pltpu.VMEM_SHARED((N_NODES, D_HID), jnp.float32),
            pltpu.VMEM_SHARED((N_NODES,), jnp.float32),
            pltpu.VMEM((CH,), jnp.int32),
            pltpu.VMEM((CH,), jnp.int32),
            pltpu.VMEM((CH, D_HID), jnp.float32),
            pltpu.VMEM((CH,), jnp.float32),
            pltpu.VMEM((SL_B, D_HID), jnp.float32),
            pltpu.VMEM((SL_B,), jnp.float32),
            pltpu.SemaphoreType.DMA,
        ],
    )
    def k(src_hbm, dst_hbm, y1_hbm, agg_out, deg_out, agg_sh, deg_sh,
          src_v, dst_v, rows_v, ones_v, ztile_v, zrow_v, sem):
        c = lax.axis_index("c")
        s = lax.axis_index("s")
        wid = s * NC + c
        zero16 = jnp.zeros((D_HID,), jnp.float32)
        one16 = jnp.ones((16,), jnp.float32)

        def zfill(i, _):
            ztile_v[i, :] = zero16
            return 0

        lax.fori_loop(0, SL_B, zfill, 0)

        def zrfill(i, _):
            zrow_v[pl.ds(i * 16, 16)] = zero16
            return 0

        lax.fori_loop(0, SL_B // 16, zrfill, 0)

        for i in range(CH // 16):
            ones_v[pl.ds(i * 16, 16)] = one16

        @pl.when(s < 15)
        def _():
            n0 = s * SL_A
            pltpu.sync_copy(ztile_v.at[pl.ds(0, SL_A)], agg_sh.at[pl.ds(n0, SL_A)])
            pltpu.sync_copy(zrow_v.at[pl.ds(0, SL_A)], deg_sh.at[pl.ds(n0, SL_A)])

        @pl.when(s == 15)
        def _():
            pltpu.sync_copy(ztile_v, agg_sh.at[pl.ds(15 * SL_A, SL_B)])
            pltpu.sync_copy(zrow_v, deg_sh.at[pl.ds(15 * SL_A, SL_B)])

        plsc.subcore_barrier()

        base = wid * EPW

        def eloop(j, _):
            off = pl.multiple_of(base + j * CH, 8)
            pltpu.sync_copy(src_hbm.at[pl.ds(off, CH)], src_v)
            pltpu.sync_copy(dst_hbm.at[pl.ds(off, CH)], dst_v)
            pltpu.async_copy(y1_hbm.at[src_v], rows_v, sem).wait()
            pltpu.sync_copy(rows_v, agg_sh.at[dst_v], add=True)
            pltpu.sync_copy(ones_v, deg_sh.at[dst_v], add=True)
            return 0

        lax.fori_loop(0, NCH, eloop, 0)
        plsc.subcore_barrier()

        @pl.when(s < 15)
        def _():
            n0 = s * SL_A
            r0 = c * N_NODES + n0
            pltpu.sync_copy(agg_sh.at[pl.ds(n0, SL_A)], ztile_v.at[pl.ds(0, SL_A)])
            pltpu.sync_copy(ztile_v.at[pl.ds(0, SL_A)], agg_out.at[pl.ds(r0, SL_A)])
            pltpu.sync_copy(deg_sh.at[pl.ds(n0, SL_A)], zrow_v.at[pl.ds(0, SL_A)])
            pltpu.sync_copy(zrow_v.at[pl.ds(0, SL_A)], deg_out.at[pl.ds(r0, SL_A)])

        @pl.when(s == 15)
        def _():
            n0 = 15 * SL_A
            r0 = c * N_NODES + n0
            pltpu.sync_copy(agg_sh.at[pl.ds(n0, SL_B)], ztile_v)
            pltpu.sync_copy(ztile_v, agg_out.at[pl.ds(r0, SL_B)])
            pltpu.sync_copy(deg_sh.at[pl.ds(n0, SL_B)], zrow_v)
            pltpu.sync_copy(zrow_v, deg_out.at[pl.ds(r0, SL_B)])

    return k(src, dst, y1)


def _sc_aggregate2(src, dst, y2):
    mesh = plsc.VectorSubcoreMesh(core_axis_name="c", subcore_axis_name="s")

    @functools.partial(
        pl.kernel,
        out_type=jax.ShapeDtypeStruct((NC * N_NODES,), jnp.float32),
        mesh=mesh,
        compiler_params=pltpu.CompilerParams(use_tc_tiling_on_sc=False),
        scratch_types=[
            pltpu.VMEM_SHARED((N_NODES,), jnp.float32),
            pltpu.VMEM((CH,), jnp.int32),
            pltpu.VMEM((CH,), jnp.int32),
            pltpu.VMEM((CH,), jnp.float32),
            pltpu.VMEM((SL_B,), jnp.float32),
            pltpu.SemaphoreType.DMA,
        ],
    )
    def k(src_hbm, dst_hbm, y2_hbm, agg_out, agg_sh, src_v, dst_v, vals_v,
          zrow_v, sem):
        c = lax.axis_index("c")
        s = lax.axis_index("s")
        wid = s * NC + c
        zero16 = jnp.zeros((16,), jnp.float32)

        def zrfill(i, _):
            zrow_v[pl.ds(i * 16, 16)] = zero16
            return 0

        lax.fori_loop(0, SL_B // 16, zrfill, 0)

        @pl.when(s < 15)
        def _():
            pltpu.sync_copy(zrow_v.at[pl.ds(0, SL_A)],
                            agg_sh.at[pl.ds(s * SL_A, SL_A)])

        @pl.when(s == 15)
        def _():
            pltpu.sync_copy(zrow_v, agg_sh.at[pl.ds(15 * SL_A, SL_B)])

        plsc.subcore_barrier()

        base = wid * EPW

        def eloop(j, _):
            off = pl.multiple_of(base + j * CH, 8)
            pltpu.sync_copy(src_hbm.at[pl.ds(off, CH)], src_v)
            pltpu.sync_copy(dst_hbm.at[pl.ds(off, CH)], dst_v)
            pltpu.async_copy(y2_hbm.at[src_v], vals_v, sem).wait()
            pltpu.sync_copy(vals_v, agg_sh.at[dst_v], add=True)
            return 0

        lax.fori_loop(0, NCH, eloop, 0)
        plsc.subcore_barrier()

        @pl.when(s < 15)
        def _():
            n0 = s * SL_A
            pltpu.sync_copy(agg_sh.at[pl.ds(n0, SL_A)], zrow_v.at[pl.ds(0, SL_A)])
            pltpu.sync_copy(zrow_v.at[pl.ds(0, SL_A)],
                            agg_out.at[pl.ds(c * N_NODES + n0, SL_A)])

        @pl.when(s == 15)
        def _():
            n0 = 15 * SL_A
            pltpu.sync_copy(agg_sh.at[pl.ds(n0, SL_B)], zrow_v)
            pltpu.sync_copy(zrow_v, agg_out.at[pl.ds(c * N_NODES + n0, SL_B)])

    return k(src, dst, y2)


def _tc_layer_mid(aggp, degp, p1, b1l, w2l, w2r, b2l):
    def body(aggp_ref, degp_ref, p1_ref, b1l_ref, w2l_ref, w2r_ref, b2l_ref,
             y2_ref, p2b_ref, dinv_ref):
        agg = aggp_ref[0:N_NODES, :] + aggp_ref[N_NODES:2 * N_NODES, :]
        deg = degp_ref[0:N_NODES, :] + degp_ref[N_NODES:2 * N_NODES, :]
        dinv = 1.0 / jnp.maximum(deg, 1.0)
        h = jnp.maximum(agg * dinv + b1l_ref[...] + p1_ref[...], 0.0)
        y2_ref[...] = jnp.sum(h * w2l_ref[...], axis=1, keepdims=True)
        p2b_ref[...] = jnp.sum(h * w2r_ref[...], axis=1, keepdims=True) + b2l_ref[...]
        dinv_ref[...] = dinv

    return pl.pallas_call(
        body,
        out_shape=[
            jax.ShapeDtypeStruct((N_NODES, 1), jnp.float32),
            jax.ShapeDtypeStruct((N_NODES, 1), jnp.float32),
            jax.ShapeDtypeStruct((N_NODES, 1), jnp.float32),
        ],
    )(aggp, degp, p1, b1l, w2l, w2r, b2l)


def _tc_final(agg2p, dinv, p2b):
    def body(a_ref, d_ref, p_ref, o_ref):
        a = a_ref[0:N_NODES, :] + a_ref[N_NODES:2 * N_NODES, :]
        o_ref[...] = a * d_ref[...] + p_ref[...]

    return pl.pallas_call(
        body,
        out_shape=jax.ShapeDtypeStruct((N_NODES, 1), jnp.float32),
    )(agg2p, dinv, p2b)


def kernel(x, edge_index, W1l, b1l, W1r, W2l, b2l, W2r):
    src = edge_index[0].astype(jnp.int32)
    dst = edge_index[1].astype(jnp.int32)

    y1, p1 = _tc_linear2(x, W1l.T, W1r.T)
    aggp, degp = _sc_aggregate1(src, dst, y1)
    y2, p2b, dinv = _tc_layer_mid(
        aggp, degp.reshape(NC * N_NODES, 1), p1,
        b1l.reshape(1, D_HID), W2l, W2r, b2l.reshape(1, 1))
    agg2p = _sc_aggregate2(src, dst, y2.reshape(-1))
    out = _tc_final(agg2p.reshape(NC * N_NODES, 1), dinv, p2b)
    return out


# speedup vs baseline: 20.3352x; 2.9105x over previous
import functools

import jax
import jax.numpy as jnp
from jax import lax
from jax.experimental import pallas as pl
from jax.experimental.pallas import tpu as pltpu
from jax.experimental.pallas import tpu_sc as plsc

N_NODES = 10000
N_EDGES = 320000
D_IN = 128
D_HID = 16

NC = 2
NS = 16
NW = NC * NS
EPW = N_EDGES // NW
CH = 2000
NCH = EPW // CH
SL_A = 624
SL_B = N_NODES - 15 * SL_A


def _tc_linear2(x, wl, wr):
    def body(x_ref, wl_ref, wr_ref, y_ref, p_ref):
        xv = x_ref[...]
        y_ref[...] = jnp.dot(xv, wl_ref[...], preferred_element_type=jnp.float32)
        p_ref[...] = jnp.dot(xv, wr_ref[...], preferred_element_type=jnp.float32)

    n = x.shape[0]
    return pl.pallas_call(
        body,
        out_shape=[
            jax.ShapeDtypeStruct((n, wl.shape[1]), jnp.float32),
            jax.ShapeDtypeStruct((n, wr.shape[1]), jnp.float32),
        ],
    )(x, wl, wr)


def _sc_aggregate1(src, dst, y1):
    mesh = plsc.VectorSubcoreMesh(core_axis_name="c", subcore_axis_name="s")

    @functools.partial(
        pl.kernel,
        out_type=[
            jax.ShapeDtypeStruct((NC * N_NODES, D_HID), jnp.float32),
            jax.ShapeDtypeStruct((NC * N_NODES,), jnp.float32),
        ],
        mesh=mesh,
        compiler_params=pltpu.CompilerParams(use_tc_tiling_on_sc=False),
        scratch_types=[
            pltpu.VMEM_SHARED((N_NODES, D_HID), jnp.float32),
            pltpu.VMEM_SHARED((N_NODES,), jnp.float32),
            pltpu.VMEM((CH,), jnp.int32),
            pltpu.VMEM((CH,), jnp.int32),
            pltpu.VMEM((CH, D_HID), jnp.float32),
            pltpu.VMEM((CH,), jnp.float32),
            pltpu.VMEM((SL_B, D_HID), jnp.float32),
            pltpu.VMEM((SL_B,), jnp.float32),
            pltpu.SemaphoreType.DMA,
        ],
    )
    def k(src_hbm, dst_hbm, y1_hbm, agg_out, deg_out, agg_sh, deg_sh,
          src_v, dst_v, rows_v, ones_v, ztile_v, zrow_v, sem):
        c = lax.axis_index("c")
        s = lax.axis_index("s")
        wid = s * NC + c
        zero16 = jnp.zeros((D_HID,), jnp.float32)
        one16 = jnp.ones((16,), jnp.float32)

        def zfill(i, _):
            ztile_v[i, :] = zero16
            return 0

        lax.fori_loop(0, SL_B, zfill, 0)

        def zrfill(i, _):
            zrow_v[pl.ds(i * 16, 16)] = zero16
            return 0

        lax.fori_loop(0, SL_B // 16, zrfill, 0)

        for i in range(CH // 16):
            ones_v[pl.ds(i * 16, 16)] = one16

        @pl.when(s < 15)
        def _():
            n0 = s * SL_A
            pltpu.sync_copy(ztile_v.at[pl.ds(0, SL_A)], agg_sh.at[pl.ds(n0, SL_A)])
            pltpu.sync_copy(zrow_v.at[pl.ds(0, SL_A)], deg_sh.at[pl.ds(n0, SL_A)])

        @pl.when(s == 15)
        def _():
            pltpu.sync_copy(ztile_v, agg_sh.at[pl.ds(15 * SL_A, SL_B)])
            pltpu.sync_copy(zrow_v, deg_sh.at[pl.ds(15 * SL_A, SL_B)])

        plsc.subcore_barrier()

        base = wid * EPW

        def eloop(j, _):
            off = pl.multiple_of(base + j * CH, 8)
            pltpu.sync_copy(src_hbm.at[pl.ds(off, CH)], src_v)
            pltpu.sync_copy(dst_hbm.at[pl.ds(off, CH)], dst_v)
            pltpu.async_copy(y1_hbm.at[src_v], rows_v, sem).wait()
            pltpu.sync_copy(rows_v, agg_sh.at[dst_v], add=True)
            pltpu.sync_copy(ones_v, deg_sh.at[dst_v], add=True)
            return 0

        lax.fori_loop(0, NCH, eloop, 0)
        plsc.subcore_barrier()

        @pl.when(s < 15)
        def _():
            n0 = s * SL_A
            r0 = c * N_NODES + n0
            pltpu.sync_copy(agg_sh.at[pl.ds(n0, SL_A)], ztile_v.at[pl.ds(0, SL_A)])
            pltpu.sync_copy(ztile_v.at[pl.ds(0, SL_A)], agg_out.at[pl.ds(r0, SL_A)])
            pltpu.sync_copy(deg_sh.at[pl.ds(n0, SL_A)], zrow_v.at[pl.ds(0, SL_A)])
            pltpu.sync_copy(zrow_v.at[pl.ds(0, SL_A)], deg_out.at[pl.ds(r0, SL_A)])

        @pl.when(s == 15)
        def _():
            n0 = 15 * SL_A
            r0 = c * N_NODES + n0
            pltpu.sync_copy(agg_sh.at[pl.ds(n0, SL_B)], ztile_v)
            pltpu.sync_copy(ztile_v, agg_out.at[pl.ds(r0, SL_B)])
            pltpu.sync_copy(deg_sh.at[pl.ds(n0, SL_B)], zrow_v)
            pltpu.sync_copy(zrow_v, deg_out.at[pl.ds(r0, SL_B)])

    return k(src, dst, y1)


def _sc_aggregate2(src, dst, y2):
    mesh = plsc.VectorSubcoreMesh(core_axis_name="c", subcore_axis_name="s")

    @functools.partial(
        pl.kernel,
        out_type=jax.ShapeDtypeStruct((NC * N_NODES,), jnp.float32),
        mesh=mesh,
        compiler_params=pltpu.CompilerParams(use_tc_tiling_on_sc=False),
        scratch_types=[
            pltpu.VMEM_SHARED((N_NODES,), jnp.float32),
            pltpu.VMEM((CH,), jnp.int32),
            pltpu.VMEM((CH,), jnp.int32),
            pltpu.VMEM((CH,), jnp.float32),
            pltpu.VMEM((SL_B,), jnp.float32),
            pltpu.SemaphoreType.DMA,
        ],
    )
    def k(src_hbm, dst_hbm, y2_hbm, agg_out, agg_sh, src_v, dst_v, vals_v,
          zrow_v, sem):
        c = lax.axis_index("c")
        s = lax.axis_index("s")
        wid = s * NC + c
        zero16 = jnp.zeros((16,), jnp.float32)

        def zrfill(i, _):
            zrow_v[pl.ds(i * 16, 16)] = zero16
            return 0

        lax.fori_loop(0, SL_B // 16, zrfill, 0)

        @pl.when(s < 15)
        def _():
            pltpu.sync_copy(zrow_v.at[pl.ds(0, SL_A)],
                            agg_sh.at[pl.ds(s * SL_A, SL_A)])

        @pl.when(s == 15)
        def _():
            pltpu.sync_copy(zrow_v, agg_sh.at[pl.ds(15 * SL_A, SL_B)])

        plsc.subcore_barrier()

        base = wid * EPW

        def eloop(j, _):
            off = pl.multiple_of(base + j * CH, 8)
            pltpu.sync_copy(src_hbm.at[pl.ds(off, CH)], src_v)
            pltpu.sync_copy(dst_hbm.at[pl.ds(off, CH)], dst_v)
            pltpu.async_copy(y2_hbm.at[src_v], vals_v, sem).wait()
            pltpu.sync_copy(vals_v, agg_sh.at[dst_v], add=True)
            return 0

        lax.fori_loop(0, NCH, eloop, 0)
        plsc.subcore_barrier()

        @pl.when(s < 15)
        def _():
            n0 = s * SL_A
            pltpu.sync_copy(agg_sh.at[pl.ds(n0, SL_A)], zrow_v.at[pl.ds(0, SL_A)])
            pltpu.sync_copy(zrow_v.at[pl.ds(0, SL_A)],
                            agg_out.at[pl.ds(c * N_NODES + n0, SL_A)])

        @pl.when(s == 15)
        def _():
            n0 = 15 * SL_A
            pltpu.sync_copy(agg_sh.at[pl.ds(n0, SL_B)], zrow_v)
            pltpu.sync_copy(zrow_v, agg_out.at[pl.ds(c * N_NODES + n0, SL_B)])

    return k(src, dst, y2)


def _tc_layer_mid(aggp, degp, p1, b1l, w2l, w2r, b2l):
    def body(aggp_ref, degp_ref, p1_ref, b1l_ref, w2l_ref, w2r_ref, b2l_ref,
             y2_ref, p2b_ref, dinv_ref):
        agg = aggp_ref[0:N_NODES, :] + aggp_ref[N_NODES:2 * N_NODES, :]
        deg = degp_ref[0:N_NODES, :] + degp_ref[N_NODES:2 * N_NODES, :]
        dinv = 1.0 / jnp.maximum(deg, 1.0)
        h = jnp.maximum(agg * dinv + b1l_ref[...] + p1_ref[...], 0.0)
        y2_ref[...] = jnp.sum(h * w2l_ref[...], axis=1, keepdims=True)
        p2b_ref[...] = jnp.sum(h * w2r_ref[...], axis=1, keepdims=True) + b2l_ref[...]
        dinv_ref[...] = dinv

    return pl.pallas_call(
        body,
        out_shape=[
            jax.ShapeDtypeStruct((N_NODES, 1), jnp.float32),
            jax.ShapeDtypeStruct((N_NODES, 1), jnp.float32),
            jax.ShapeDtypeStruct((N_NODES, 1), jnp.float32),
        ],
    )(aggp, degp, p1, b1l, w2l, w2r, b2l)


def _tc_final(agg2p, dinv, p2b):
    def body(a_ref, d_ref, p_ref, o_ref):
        a = a_ref[0:N_NODES, :] + a_ref[N_NODES:2 * N_NODES, :]
        o_ref[...] = a * d_ref[...] + p_ref[...]

    return pl.pallas_call(
        body,
        out_shape=jax.ShapeDtypeStruct((N_NODES, 1), jnp.float32),
    )(agg2p, dinv, p2b)


def kernel(x, edge_index, W1l, b1l, W1r, W2l, b2l, W2r):
    src = edge_index[0].astype(jnp.int32)
    dst = edge_index[1].astype(jnp.int32)

    y1, p1 = _tc_linear2(x, W1l.T, W1r.T)
    aggp, degp = _sc_aggregate1(src, dst, y1)
    y2, p2b, dinv = _tc_layer_mid(
        aggp, degp.reshape(NC * N_NODES, 1), p1,
        b1l.reshape(1, D_HID), W2l, W2r, b2l.reshape(1, 1))
    agg2p = _sc_aggregate2(src, dst, y2.reshape(-1))
    out = _tc_final(agg2p.reshape(NC * N_NODES, 1), dinv, p2b)
    return out


# speedup vs baseline: 21.1043x; 1.0378x over previous
import functools

import jax
import jax.numpy as jnp
from jax import lax
from jax.experimental import pallas as pl
from jax.experimental.pallas import tpu as pltpu
from jax.experimental.pallas import tpu_sc as plsc

N_NODES = 10000
N_EDGES = 320000
D_IN = 128
D_HID = 16

NC = 2
NS = 16
NW = NC * NS
EPW = N_EDGES // NW
CH = 5000
NCH = EPW // CH
SL_A = 624
SL_B = N_NODES - 15 * SL_A


def _tc_linear2(x, wl, wr):
    def body(x_ref, wl_ref, wr_ref, y_ref, p_ref):
        xv = x_ref[...]
        y_ref[...] = jnp.dot(xv, wl_ref[...], preferred_element_type=jnp.float32)
        p_ref[...] = jnp.dot(xv, wr_ref[...], preferred_element_type=jnp.float32)

    n = x.shape[0]
    return pl.pallas_call(
        body,
        out_shape=[
            jax.ShapeDtypeStruct((n, wl.shape[1]), jnp.float32),
            jax.ShapeDtypeStruct((n, wr.shape[1]), jnp.float32),
        ],
    )(x, wl, wr)


def _sc_aggregate1(src, dst, y1):
    mesh = plsc.VectorSubcoreMesh(core_axis_name="c", subcore_axis_name="s")

    @functools.partial(
        pl.kernel,
        out_type=[
            jax.ShapeDtypeStruct((NC * N_NODES, D_HID), jnp.float32),
            jax.ShapeDtypeStruct((NC * N_NODES,), jnp.float32),
        ],
        mesh=mesh,
        compiler_params=pltpu.CompilerParams(use_tc_tiling_on_sc=False),
        scratch_types=[
            pltpu.VMEM_SHARED((N_NODES, D_HID), jnp.float32),
            pltpu.VMEM_SHARED((N_NODES,), jnp.float32),
            pltpu.VMEM((CH,), jnp.int32),
            pltpu.VMEM((CH,), jnp.int32),
            pltpu.VMEM((CH, D_HID), jnp.float32),
            pltpu.VMEM((CH,), jnp.float32),
            pltpu.VMEM((SL_B, D_HID), jnp.float32),
            pltpu.VMEM((SL_B,), jnp.float32),
            pltpu.SemaphoreType.DMA,
        ],
    )
    def k(src_hbm, dst_hbm, y1_hbm, agg_out, deg_out, agg_sh, deg_sh,
          src_v, dst_v, rows_v, ones_v, ztile_v, zrow_v, sem):
        c = lax.axis_index("c")
        s = lax.axis_index("s")
        wid = s * NC + c
        zero16 = jnp.zeros((D_HID,), jnp.float32)
        one16 = jnp.ones((16,), jnp.float32)

        def zfill(i, _):
            ztile_v[i, :] = zero16
            return 0

        lax.fori_loop(0, SL_B, zfill, 0)

        def zrfill(i, _):
            zrow_v[pl.ds(i * 16, 16)] = zero16
            return 0

        lax.fori_loop(0, SL_B // 16, zrfill, 0)

        for i in range(CH // 16):
            ones_v[pl.ds(i * 16, 16)] = one16

        @pl.when(s < 15)
        def _():
            n0 = s * SL_A
            pltpu.sync_copy(ztile_v.at[pl.ds(0, SL_A)], agg_sh.at[pl.ds(n0, SL_A)])
            pltpu.sync_copy(zrow_v.at[pl.ds(0, SL_A)], deg_sh.at[pl.ds(n0, SL_A)])

        @pl.when(s == 15)
        def _():
            pltpu.sync_copy(ztile_v, agg_sh.at[pl.ds(15 * SL_A, SL_B)])
            pltpu.sync_copy(zrow_v, deg_sh.at[pl.ds(15 * SL_A, SL_B)])

        plsc.subcore_barrier()

        base = wid * EPW

        def eloop(j, _):
            off = pl.multiple_of(base + j * CH, 8)
            pltpu.sync_copy(src_hbm.at[pl.ds(off, CH)], src_v)
            pltpu.sync_copy(dst_hbm.at[pl.ds(off, CH)], dst_v)
            pltpu.async_copy(y1_hbm.at[src_v], rows_v, sem).wait()
            pltpu.sync_copy(rows_v, agg_sh.at[dst_v], add=True)
            pltpu.sync_copy(ones_v, deg_sh.at[dst_v], add=True)
            return 0

        lax.fori_loop(0, NCH, eloop, 0)
        plsc.subcore_barrier()

        @pl.when(s < 15)
        def _():
            n0 = s * SL_A
            r0 = c * N_NODES + n0
            pltpu.sync_copy(agg_sh.at[pl.ds(n0, SL_A)], ztile_v.at[pl.ds(0, SL_A)])
            pltpu.sync_copy(ztile_v.at[pl.ds(0, SL_A)], agg_out.at[pl.ds(r0, SL_A)])
            pltpu.sync_copy(deg_sh.at[pl.ds(n0, SL_A)], zrow_v.at[pl.ds(0, SL_A)])
            pltpu.sync_copy(zrow_v.at[pl.ds(0, SL_A)], deg_out.at[pl.ds(r0, SL_A)])

        @pl.when(s == 15)
        def _():
            n0 = 15 * SL_A
            r0 = c * N_NODES + n0
            pltpu.sync_copy(agg_sh.at[pl.ds(n0, SL_B)], ztile_v)
            pltpu.sync_copy(ztile_v, agg_out.at[pl.ds(r0, SL_B)])
            pltpu.sync_copy(deg_sh.at[pl.ds(n0, SL_B)], zrow_v)
            pltpu.sync_copy(zrow_v, deg_out.at[pl.ds(r0, SL_B)])

    return k(src, dst, y1)


def _sc_aggregate2(src, dst, y2):
    mesh = plsc.VectorSubcoreMesh(core_axis_name="c", subcore_axis_name="s")

    @functools.partial(
        pl.kernel,
        out_type=jax.ShapeDtypeStruct((NC * N_NODES,), jnp.float32),
        mesh=mesh,
        compiler_params=pltpu.CompilerParams(use_tc_tiling_on_sc=False),
        scratch_types=[
            pltpu.VMEM_SHARED((N_NODES,), jnp.float32),
            pltpu.VMEM((CH,), jnp.int32),
            pltpu.VMEM((CH,), jnp.int32),
            pltpu.VMEM((CH,), jnp.float32),
            pltpu.VMEM((SL_B,), jnp.float32),
            pltpu.SemaphoreType.DMA,
        ],
    )
    def k(src_hbm, dst_hbm, y2_hbm, agg_out, agg_sh, src_v, dst_v, vals_v,
          zrow_v, sem):
        c = lax.axis_index("c")
        s = lax.axis_index("s")
        wid = s * NC + c
        zero16 = jnp.zeros((16,), jnp.float32)

        def zrfill(i, _):
            zrow_v[pl.ds(i * 16, 16)] = zero16
            return 0

        lax.fori_loop(0, SL_B // 16, zrfill, 0)

        @pl.when(s < 15)
        def _():
            pltpu.sync_copy(zrow_v.at[pl.ds(0, SL_A)],
                            agg_sh.at[pl.ds(s * SL_A, SL_A)])

        @pl.when(s == 15)
        def _():
            pltpu.sync_copy(zrow_v, agg_sh.at[pl.ds(15 * SL_A, SL_B)])

        plsc.subcore_barrier()

        base = wid * EPW

        def eloop(j, _):
            off = pl.multiple_of(base + j * CH, 8)
            pltpu.sync_copy(src_hbm.at[pl.ds(off, CH)], src_v)
            pltpu.sync_copy(dst_hbm.at[pl.ds(off, CH)], dst_v)
            pltpu.async_copy(y2_hbm.at[src_v], vals_v, sem).wait()
            pltpu.sync_copy(vals_v, agg_sh.at[dst_v], add=True)
            return 0

        lax.fori_loop(0, NCH, eloop, 0)
        plsc.subcore_barrier()

        @pl.when(s < 15)
        def _():
            n0 = s * SL_A
            pltpu.sync_copy(agg_sh.at[pl.ds(n0, SL_A)], zrow_v.at[pl.ds(0, SL_A)])
            pltpu.sync_copy(zrow_v.at[pl.ds(0, SL_A)],
                            agg_out.at[pl.ds(c * N_NODES + n0, SL_A)])

        @pl.when(s == 15)
        def _():
            n0 = 15 * SL_A
            pltpu.sync_copy(agg_sh.at[pl.ds(n0, SL_B)], zrow_v)
            pltpu.sync_copy(zrow_v, agg_out.at[pl.ds(c * N_NODES + n0, SL_B)])

    return k(src, dst, y2)


def _tc_layer_mid(aggp, degp, p1, b1l, w2l, w2r, b2l):
    def body(aggp_ref, degp_ref, p1_ref, b1l_ref, w2l_ref, w2r_ref, b2l_ref,
             y2_ref, p2b_ref, dinv_ref):
        agg = aggp_ref[0:N_NODES, :] + aggp_ref[N_NODES:2 * N_NODES, :]
        deg = degp_ref[0:N_NODES, :] + degp_ref[N_NODES:2 * N_NODES, :]
        dinv = 1.0 / jnp.maximum(deg, 1.0)
        h = jnp.maximum(agg * dinv + b1l_ref[...] + p1_ref[...], 0.0)
        y2_ref[...] = jnp.sum(h * w2l_ref[...], axis=1, keepdims=True)
        p2b_ref[...] = jnp.sum(h * w2r_ref[...], axis=1, keepdims=True) + b2l_ref[...]
        dinv_ref[...] = dinv

    return pl.pallas_call(
        body,
        out_shape=[
            jax.ShapeDtypeStruct((N_NODES, 1), jnp.float32),
            jax.ShapeDtypeStruct((N_NODES, 1), jnp.float32),
            jax.ShapeDtypeStruct((N_NODES, 1), jnp.float32),
        ],
    )(aggp, degp, p1, b1l, w2l, w2r, b2l)


def _tc_final(agg2p, dinv, p2b):
    def body(a_ref, d_ref, p_ref, o_ref):
        a = a_ref[0:N_NODES, :] + a_ref[N_NODES:2 * N_NODES, :]
        o_ref[...] = a * d_ref[...] + p_ref[...]

    return pl.pallas_call(
        body,
        out_shape=jax.ShapeDtypeStruct((N_NODES, 1), jnp.float32),
    )(agg2p, dinv, p2b)


def kernel(x, edge_index, W1l, b1l, W1r, W2l, b2l, W2r):
    src = edge_index[0].astype(jnp.int32)
    dst = edge_index[1].astype(jnp.int32)

    y1, p1 = _tc_linear2(x, W1l.T, W1r.T)
    aggp, degp = _sc_aggregate1(src, dst, y1)
    y2, p2b, dinv = _tc_layer_mid(
        aggp, degp.reshape(NC * N_NODES, 1), p1,
        b1l.reshape(1, D_HID), W2l, W2r, b2l.reshape(1, 1))
    agg2p = _sc_aggregate2(src, dst, y2.reshape(-1))
    out = _tc_final(agg2p.reshape(NC * N_NODES, 1), dinv, p2b)
    return out
